# Initial kernel scaffold; baseline (speedup 1.0000x reference)
import jax
import jax.numpy as jnp
from jax.experimental import pallas as pl


def kernel(x, params):
    raise NotImplementedError("write your pallas kernel here")


# speedup vs baseline: 10.9387x; 10.9387x over previous
import functools

import numpy as np
import jax
import jax.numpy as jnp
from jax.experimental import pallas as pl

_NUM_CLASSES = 10
_K = 8
_KSIZE = 3.66
_NUM_LAYERS = 6
_TOTAL_DS = 1.0 / 256
_NBHD = int(np.round(_KSIZE ** 2))
_DS = _TOTAL_DS ** (1.0 / _NUM_LAYERS)
_CHS = np.round(
    np.exp(np.linspace(np.log(16), np.log(64 * _K), _NUM_LAYERS + 1))
).astype(int)

_SENTINEL = 1.0e6


def _rup(v, k):
    return (v + k - 1) // k * k


def _layer_dims():
    dims = []
    n = 64 * 64
    for l in range(_NUM_LAYERS):
        cin = int(_CHS[l])
        cout = int(_CHS[l + 1])
        cmid = max(cout // 4, 1)
        m = int(round(n * _DS))
        dims.append((cin, cout, cmid, n, m))
        n = m
    return dims


def _bneck_kernel(
    coords_ref, vals_ref,
    w1_ref, b1_ref, wn1_ref, bn1_ref, wn2_ref, bn2_ref,
    w2_ref, b2_ref, wsc_ref, bsc_ref,
    out_c_ref, out_v_ref,
    *, npad, qb, m_real, final):
    q = pl.program_id(1)
    coords = coords_ref[0]
    vals = vals_ref[0]

    f32 = jnp.float32
    g = jax.nn.relu(
        jax.lax.dot_general(w1_ref[...], vals, (((1,), (0,)), ((), ())),
                            preferred_element_type=f32) + b1_ref[...])

    qt = coords_ref[0, :, pl.ds(q * qb, qb)]
    vq = vals_ref[0, :, pl.ds(q * qb, qb)]

    c2 = coords * coords
    cn = jax.lax.dot_general(c2, jnp.ones((2, 1), f32),
                             (((0,), (0,)), ((), ())),
                             preferred_element_type=f32)
    q2 = qt * qt
    qn = q2[0:1, :] + q2[1:2, :]
    dot = jax.lax.dot_general(coords, qt, (((0,), (0,)), ((), ())),
                              preferred_element_type=f32)
    d2t = (qn + cn) - 2.0 * dot

    gsrc = jnp.concatenate([coords, g], axis=0)
    ii = jax.lax.broadcasted_iota(jnp.int32, (npad, qb), 0)
    cmidp = g.shape[0]

    def body(j, carry):
        d2, agg = carry
        colmin = jnp.min(d2, axis=0, keepdims=True)
        idx = jnp.min(jnp.where(d2 == colmin, ii, npad), axis=0,
                      keepdims=True)
        sel = ii == idx
        oh = sel.astype(f32)
        gath = jax.lax.dot_general(gsrc, oh, (((1,), (0,)), ((), ())),
                                   preferred_element_type=f32)
        delta = gath[0:2, :] - qt
        h = jax.nn.relu(
            jax.lax.dot_general(wn1_ref[...], delta, (((1,), (0,)), ((), ())),
                                preferred_element_type=f32) + bn1_ref[...])
        wj = jax.lax.dot_general(wn2_ref[...], h, (((1,), (0,)), ((), ())),
                                 preferred_element_type=f32) + bn2_ref[...]
        agg = agg + wj * gath[2:, :]
        d2 = jnp.where(sel, jnp.inf, d2)
        return d2, agg

    agg0 = jnp.zeros((cmidp, qb), f32)
    _, agg = jax.lax.fori_loop(0, _NBHD, body, (d2t, agg0))
    agg = agg * (1.0 / _NBHD)

    out = jax.nn.relu(
        jax.lax.dot_general(w2_ref[...], agg, (((1,), (0,)), ((), ())),
                            preferred_element_type=f32) + b2_ref[...])
    sc = jax.lax.dot_general(wsc_ref[...], vq, (((1,), (0,)), ((), ())),
                             preferred_element_type=f32) + bsc_ref[...]
    res = out + sc

    lane = jax.lax.broadcasted_iota(jnp.int32, (2, qb), 1) + q * qb
    out_c_ref[0] = jnp.where(lane < m_real, qt, _SENTINEL)
    if final:
        out_v_ref[0] = res
    else:
        out_v_ref[0] = res


def _head_kernel(vals_ref, wf_ref, bf_ref, out_ref, *, m_real):
    vals = vals_ref[0]
    f32 = jnp.float32
    qb = vals.shape[1]
    lane = jax.lax.broadcasted_iota(jnp.int32, (1, qb), 1)
    mask = (lane < m_real).astype(f32)
    pooled = jax.lax.dot_general(mask, vals, (((1,), (1,)), ((), ())),
                                 preferred_element_type=f32) * (1.0 / m_real)
    logits = jax.lax.dot_general(pooled, wf_ref[...], (((1,), (1,)), ((), ())),
                                 preferred_element_type=f32) + bf_ref[...]
    out_ref[0] = logits


def _pad2(w, r, c):
    return jnp.pad(w, ((0, r - w.shape[0]), (0, c - w.shape[1])))


def _col(b, r):
    return jnp.pad(b, (0, r - b.shape[0])).reshape(r, 1)


def kernel(x, params):
    bs, _, h, w = x.shape
    n0 = h * w
    f32 = jnp.float32

    yy, xx = jnp.meshgrid(jnp.linspace(-1.0, 1.0, h),
                          jnp.linspace(-1.0, 1.0, w), indexing='ij')
    coords2 = jnp.stack([yy, xx], axis=-1).reshape(n0, 2).T
    coords = jnp.broadcast_to(coords2[None], (bs, 2, n0)).astype(f32)

    w0 = params['W0']
    b0 = params['b0']
    p0 = params['layers'][0]
    dims = _layer_dims()

    xr = x.reshape(bs, 3, n0)
    vals = jnp.pad(xr, ((0, 0), (0, 8 - 3), (0, 0)))

    for l, (cin, cout, cmid, n, m) in enumerate(dims):
        p = params['layers'][l]
        if l == 0:
            w1 = p['W1'] @ w0
            b1 = p['W1'] @ b0 + p['b1']
            wsc = p['Wsc'] @ w0
            bsc = p['Wsc'] @ b0 + p['bsc']
            cin_eff = 3
        else:
            w1, b1, wsc, bsc = p['W1'], p['b1'], p['Wsc'], p['bsc']
            cin_eff = cin
        cinp = _rup(cin_eff, 8)
        cmidp = _rup(cmid, 8)
        coutp = _rup(cout, 8)
        npad = vals.shape[2]
        mpad = _rup(m, 128)
        if npad == 4096:
            qb = 512
            mpad = _rup(m, qb)
        else:
            qb = mpad
        nqb = mpad // qb

        w1p = _pad2(w1, cmidp, cinp)
        b1p = _col(b1, cmidp)
        wn1p = p['Wn1']
        bn1p = _col(p['bn1'], 32)
        wn2p = _pad2(p['Wn2'], cmidp, 32)
        bn2p = _col(p['bn2'], cmidp)
        w2p = _pad2(p['W2'], coutp, cmidp)
        b2p = _col(p['b2'], coutp)
        wscp = _pad2(wsc, coutp, cinp)
        bscp = _col(bsc, coutp)

        kfn = functools.partial(_bneck_kernel, npad=npad, qb=qb,
                                m_real=m, final=(l == _NUM_LAYERS - 1))
        bcast = lambda *_: (0, 0)
        grid = (bs, nqb)
        coords, vals = pl.pallas_call(
            kfn,
            grid=grid,
            in_specs=[
                pl.BlockSpec((1, 2, npad), lambda b, q: (b, 0, 0)),
                pl.BlockSpec((1, cinp, npad), lambda b, q: (b, 0, 0)),
                pl.BlockSpec((cmidp, cinp), bcast),
                pl.BlockSpec((cmidp, 1), bcast),
                pl.BlockSpec((32, 2), bcast),
                pl.BlockSpec((32, 1), bcast),
                pl.BlockSpec((cmidp, 32), bcast),
                pl.BlockSpec((cmidp, 1), bcast),
                pl.BlockSpec((coutp, cmidp), bcast),
                pl.BlockSpec((coutp, 1), bcast),
                pl.BlockSpec((coutp, cinp), bcast),
                pl.BlockSpec((coutp, 1), bcast),
            ],
            out_specs=[
                pl.BlockSpec((1, 2, qb), lambda b, q: (b, 0, q)),
                pl.BlockSpec((1, coutp, qb), lambda b, q: (b, 0, q)),
            ],
            out_shape=[
                jax.ShapeDtypeStruct((bs, 2, mpad), f32),
                jax.ShapeDtypeStruct((bs, coutp, mpad), f32),
            ],
        )(coords, vals, w1p, b1p, wn1p, bn1p, wn2p, bn2p, w2p, b2p,
          wscp, bscp)

    m_last = dims[-1][4]
    coutp = vals.shape[1]
    qb = vals.shape[2]
    wfp = _pad2(params['Wf'], 16, coutp)
    bfp = jnp.pad(params['bf'], (0, 16 - _NUM_CLASSES)).reshape(1, 16)
    hfn = functools.partial(_head_kernel, m_real=m_last)
    logits = pl.pallas_call(
        hfn,
        grid=(bs,),
        in_specs=[
            pl.BlockSpec((1, coutp, qb), lambda b: (b, 0, 0)),
            pl.BlockSpec((16, coutp), lambda b: (0, 0)),
            pl.BlockSpec((1, 16), lambda b: (0, 0)),
        ],
        out_specs=pl.BlockSpec((1, 1, 16), lambda b: (b, 0, 0)),
        out_shape=jax.ShapeDtypeStruct((bs, 1, 16), f32),
    )(vals, wfp, bfp)
    return logits.reshape(bs, 16)[:, :_NUM_CLASSES]


# speedup vs baseline: 104.3430x; 9.5389x over previous
import functools

import numpy as np
import jax
import jax.numpy as jnp
from jax.experimental import pallas as pl

_NUM_CLASSES = 10
_K = 8
_KSIZE = 3.66
_NUM_LAYERS = 6
_TOTAL_DS = 1.0 / 256
_NBHD = int(np.round(_KSIZE ** 2))
_DS = _TOTAL_DS ** (1.0 / _NUM_LAYERS)
_CHS = np.round(
    np.exp(np.linspace(np.log(16), np.log(64 * _K), _NUM_LAYERS + 1))
).astype(int)


def _rup(v, k):
    return (v + k - 1) // k * k


def _layer_dims():
    dims = []
    n = 64 * 64
    for l in range(_NUM_LAYERS):
        cin = int(_CHS[l])
        cout = int(_CHS[l + 1])
        cmid = max(cout // 4, 1)
        m = int(round(n * _DS))
        dims.append((cin, cout, cmid, n, m))
        n = m
    return dims


@functools.lru_cache(maxsize=2)
def _knn_structure(h, w):
    lin_y = np.linspace(-1.0, 1.0, h).astype(np.float32)
    lin_x = np.linspace(-1.0, 1.0, w).astype(np.float32)
    yy, xx = np.meshgrid(lin_y, lin_x, indexing='ij')
    pts = np.stack([yy.reshape(-1), xx.reshape(-1)], axis=1).astype(np.float32)

    out = []
    for (cin, cout, cmid, n, m) in _layer_dims():
        ct = pts[:n]
        qt = pts[:m]
        py = (qt[:, 0:1] * ct[None, :, 0]).astype(np.float32).reshape(m, n)
        px = (qt[:, 1:2] * ct[None, :, 1]).astype(np.float32).reshape(m, n)
        dot = py + px
        cn = (ct[:, 0] * ct[:, 0] + ct[:, 1] * ct[:, 1]).astype(np.float32)
        qn = (qt[:, 0] * qt[:, 0] + qt[:, 1] * qt[:, 1]).astype(np.float32)
        d2 = (qn[:, None] + cn[None, :]) - np.float32(2.0) * dot
        idx = np.argsort(d2, axis=1, kind='stable')[:, :_NBHD]
        nbc = ct[idx]
        delta = (nbc - qt[:, None, :]).astype(np.float32)
        out.append((idx.astype(np.int32), delta))
    return out


def _bneck_kernel(
    vals_ref, lidx_ref, delta_ref,
    w1_ref, b1_ref, wn1_ref, bn1_ref, wn2_ref, bn2_ref,
    w2_ref, b2_ref, wsc_ref, bsc_ref,
    out_ref, *, npad, qb, win, margin):
    q = pl.program_id(1)
    f32 = jnp.float32

    ws = jnp.minimum(jnp.maximum(q * qb - margin, 0), npad - win)
    ws = pl.multiple_of(ws, margin)
    src = vals_ref[0, :, pl.ds(ws, win)]
    vq = vals_ref[0, :, pl.ds(q * qb, qb)]

    g = jax.nn.relu(
        jax.lax.dot_general(w1_ref[...], src, (((1,), (0,)), ((), ())),
                            preferred_element_type=f32) + b1_ref[...])

    lidx = lidx_ref[0]
    delta = delta_ref[0]
    ii = jax.lax.broadcasted_iota(jnp.int32, (win, qb), 0)
    cmidp = g.shape[0]
    agg = jnp.zeros((cmidp, qb), f32)
    for j in range(_NBHD):
        oh = (ii == lidx[j:j + 1, :]).astype(f32)
        nbf = jax.lax.dot_general(g, oh, (((1,), (0,)), ((), ())),
                                  preferred_element_type=f32)
        dj = delta[2 * j:2 * j + 2, :]
        hh = jax.nn.relu(
            jax.lax.dot_general(wn1_ref[...], dj, (((1,), (0,)), ((), ())),
                                preferred_element_type=f32) + bn1_ref[...])
        wj = jax.lax.dot_general(wn2_ref[...], hh, (((1,), (0,)), ((), ())),
                                 preferred_element_type=f32) + bn2_ref[...]
        agg = agg + wj * nbf
    agg = agg * (1.0 / _NBHD)

    out = jax.nn.relu(
        jax.lax.dot_general(w2_ref[...], agg, (((1,), (0,)), ((), ())),
                            preferred_element_type=f32) + b2_ref[...])
    sc = jax.lax.dot_general(wsc_ref[...], vq, (((1,), (0,)), ((), ())),
                             preferred_element_type=f32) + bsc_ref[...]
    out_ref[0] = out + sc


def _head_kernel(vals_ref, wf_ref, bf_ref, out_ref, *, m_real):
    vals = vals_ref[0]
    f32 = jnp.float32
    qb = vals.shape[1]
    lane = jax.lax.broadcasted_iota(jnp.int32, (1, qb), 1)
    mask = (lane < m_real).astype(f32)
    pooled = jax.lax.dot_general(mask, vals, (((1,), (1,)), ((), ())),
                                 preferred_element_type=f32) * (1.0 / m_real)
    logits = jax.lax.dot_general(pooled, wf_ref[...], (((1,), (1,)), ((), ())),
                                 preferred_element_type=f32) + bf_ref[...]
    out_ref[0] = logits


def _pad2(w, r, c):
    return jnp.pad(w, ((0, r - w.shape[0]), (0, c - w.shape[1])))


def _col(b, r):
    return jnp.pad(b, (0, r - b.shape[0])).reshape(r, 1)


def kernel(x, params):
    bs, _, h, w = x.shape
    n0 = h * w
    f32 = jnp.float32

    knn = _knn_structure(h, w)
    dims = _layer_dims()

    w0 = params['W0']
    b0 = params['b0']

    xr = x.reshape(bs, 3, n0)
    vals = jnp.pad(xr, ((0, 0), (0, 8 - 3), (0, 0)))

    for l, (cin, cout, cmid, n, m) in enumerate(dims):
        p = params['layers'][l]
        if l == 0:
            w1 = p['W1'] @ w0
            b1 = p['W1'] @ b0 + p['b1']
            wsc = p['Wsc'] @ w0
            bsc = p['Wsc'] @ b0 + p['bsc']
            cin_eff = 3
        else:
            w1, b1, wsc, bsc = p['W1'], p['b1'], p['Wsc'], p['bsc']
            cin_eff = cin
        cinp = _rup(cin_eff, 8)
        cmidp = _rup(cmid, 8)
        coutp = _rup(cout, 8)
        npad = vals.shape[2]
        if npad >= 2048:
            qb = 512
        else:
            qb = _rup(m, 128)
        mpad = _rup(m, qb)
        nqb = mpad // qb
        margin = 256
        win = min(npad, qb + 2 * margin)

        idx_np, delta_np = knn[l]
        lidx_np = np.full((nqb, _NBHD, qb), -1, np.int32)
        delta_p = np.zeros((nqb, 2 * _NBHD, qb), np.float32)
        for qi in range(nqb):
            lo = qi * qb
            hi = min(m, lo + qb)
            if hi <= lo:
                continue
            ws = min(max(lo - margin, 0), npad - win)
            lidx_np[qi, :, :hi - lo] = (idx_np[lo:hi] - ws).T
            for j in range(_NBHD):
                delta_p[qi, 2 * j:2 * j + 2, :hi - lo] = delta_np[lo:hi, j].T
        lidx = jnp.asarray(lidx_np)
        deltas = jnp.asarray(delta_p)

        w1p = _pad2(w1, cmidp, cinp)
        b1p = _col(b1, cmidp)
        wn1p = p['Wn1']
        bn1p = _col(p['bn1'], 32)
        wn2p = _pad2(p['Wn2'], cmidp, 32)
        bn2p = _col(p['bn2'], cmidp)
        w2p = _pad2(p['W2'], coutp, cmidp)
        b2p = _col(p['b2'], coutp)
        wscp = _pad2(wsc, coutp, cinp)
        bscp = _col(bsc, coutp)

        kfn = functools.partial(_bneck_kernel, npad=npad, qb=qb,
                                win=win, margin=margin)
        bcast = lambda *_: (0, 0)
        vals = pl.pallas_call(
            kfn,
            grid=(bs, nqb),
            in_specs=[
                pl.BlockSpec((1, cinp, npad), lambda b, q: (b, 0, 0)),
                pl.BlockSpec((1, _NBHD, qb), lambda b, q: (q, 0, 0)),
                pl.BlockSpec((1, 2 * _NBHD, qb), lambda b, q: (q, 0, 0)),
                pl.BlockSpec((cmidp, cinp), bcast),
                pl.BlockSpec((cmidp, 1), bcast),
                pl.BlockSpec((32, 2), bcast),
                pl.BlockSpec((32, 1), bcast),
                pl.BlockSpec((cmidp, 32), bcast),
                pl.BlockSpec((cmidp, 1), bcast),
                pl.BlockSpec((coutp, cmidp), bcast),
                pl.BlockSpec((coutp, 1), bcast),
                pl.BlockSpec((coutp, cinp), bcast),
                pl.BlockSpec((coutp, 1), bcast),
            ],
            out_specs=pl.BlockSpec((1, coutp, qb), lambda b, q: (b, 0, q)),
            out_shape=jax.ShapeDtypeStruct((bs, coutp, mpad), f32),
        )(vals, lidx, deltas, w1p, b1p, wn1p, bn1p, wn2p, bn2p, w2p, b2p,
          wscp, bscp)

    m_last = dims[-1][4]
    coutp = vals.shape[1]
    qb = vals.shape[2]
    wfp = _pad2(params['Wf'], 16, coutp)
    bfp = jnp.pad(params['bf'], (0, 16 - _NUM_CLASSES)).reshape(1, 16)
    hfn = functools.partial(_head_kernel, m_real=m_last)
    logits = pl.pallas_call(
        hfn,
        grid=(bs,),
        in_specs=[
            pl.BlockSpec((1, coutp, qb), lambda b: (b, 0, 0)),
            pl.BlockSpec((16, coutp), lambda b: (0, 0)),
            pl.BlockSpec((1, 16), lambda b: (0, 0)),
        ],
        out_specs=pl.BlockSpec((1, 1, 16), lambda b: (b, 0, 0)),
        out_shape=jax.ShapeDtypeStruct((bs, 1, 16), f32),
    )(vals, wfp, bfp)
    return logits.reshape(bs, 16)[:, :_NUM_CLASSES]


# speedup vs baseline: 181.8133x; 1.7425x over previous
import functools

import numpy as np
import jax
import jax.numpy as jnp
from jax.experimental import pallas as pl

_NUM_CLASSES = 10
_K = 8
_KSIZE = 3.66
_NUM_LAYERS = 6
_TOTAL_DS = 1.0 / 256
_NBHD = int(np.round(_KSIZE ** 2))
_DS = _TOTAL_DS ** (1.0 / _NUM_LAYERS)
_CHS = np.round(
    np.exp(np.linspace(np.log(16), np.log(64 * _K), _NUM_LAYERS + 1))
).astype(int)

_MARG = 256


def _rup(v, k):
    return (v + k - 1) // k * k


def _layer_dims():
    dims = []
    n = 64 * 64
    for l in range(_NUM_LAYERS):
        cin = int(_CHS[l])
        cout = int(_CHS[l + 1])
        cmid = max(cout // 4, 1)
        m = int(round(n * _DS))
        dims.append((cin, cout, cmid, n, m))
        n = m
    return dims


def _round_bf16(x):
    u = x.astype(np.float32).view(np.uint32)
    bias = np.uint32(0x7FFF) + ((u >> np.uint32(16)) & np.uint32(1))
    return ((u + bias) & np.uint32(0xFFFF0000)).view(np.float32)


@functools.lru_cache(maxsize=2)
def _stencil_structure(h, w):
    lin_y = np.linspace(-1.0, 1.0, h).astype(np.float32)
    lin_x = np.linspace(-1.0, 1.0, w).astype(np.float32)
    yy, xx = np.meshgrid(lin_y, lin_x, indexing='ij')
    pts = np.stack([yy.reshape(-1), xx.reshape(-1)], axis=1).astype(np.float32)

    out = []
    for (cin, cout, cmid, n, m) in _layer_dims():
        ct = pts[:n]
        qt = pts[:m]
        qb16 = _round_bf16(qt)
        cb16 = _round_bf16(ct)
        py = (qb16[:, 0:1] * cb16[None, :, 0]).astype(np.float32).reshape(m, n)
        px = (qb16[:, 1:2] * cb16[None, :, 1]).astype(np.float32).reshape(m, n)
        dot = py + px
        cn = (ct[:, 0] * ct[:, 0] + ct[:, 1] * ct[:, 1]).astype(np.float32)
        qn = (qt[:, 0] * qt[:, 0] + qt[:, 1] * qt[:, 1]).astype(np.float32)
        d2 = (qn[:, None] + cn[None, :]) - np.float32(2.0) * dot
        idx = np.argsort(d2, axis=1, kind='stable')[:, :_NBHD]

        off = idx - np.arange(m)[:, None]
        offs = np.unique(off)
        omap = {int(o): i for i, o in enumerate(offs)}
        mpad = _rup(m, 128)
        mask = np.zeros((len(offs), mpad), np.float32)
        dtab = np.zeros((2, len(offs)), np.float32)
        dseen = np.zeros(len(offs), bool)
        for q in range(m):
            for j in range(_NBHD):
                o = int(off[q, j])
                i = omap[o]
                mask[i, q] = 1.0
                if not dseen[i]:
                    dtab[:, i] = ct[idx[q, j]] - qt[q]
                    dseen[i] = True
        out.append((tuple(int(o) for o in offs), mask, dtab))
    return out


def _fused_kernel(*refs, dims, offsets):
    f32 = jnp.float32
    x_ref, w0_ref, b0_ref = refs[0], refs[1], refs[2]
    vals = jax.lax.dot_general(w0_ref[...], x_ref[0], (((1,), (0,)), ((), ())),
                               preferred_element_type=f32) + b0_ref[...]
    pos = 3
    for l, (cin, cout, cmid, n, m) in enumerate(dims):
        (mask_ref, dtab_ref, w1_ref, b1_ref, wn1_ref, bn1_ref, wn2_ref,
         bn2_ref, w2_ref, b2_ref, wsc_ref, bsc_ref) = refs[pos:pos + 12]
        pos += 12
        offs = offsets[l]
        mpad = mask_ref.shape[1]
        npad = vals.shape[1]
        cmidp = w1_ref.shape[0]

        g = jax.nn.relu(
            jax.lax.dot_general(w1_ref[...], vals, (((1,), (0,)), ((), ())),
                                preferred_element_type=f32) + b1_ref[...])
        gp = jnp.concatenate(
            [jnp.zeros((cmidp, _MARG), f32), g, jnp.zeros((cmidp, _MARG), f32)],
            axis=1)

        hh = jax.nn.relu(
            jax.lax.dot_general(wn1_ref[...], dtab_ref[...],
                                (((1,), (0,)), ((), ())),
                                preferred_element_type=f32) + bn1_ref[...])
        wtab = jax.lax.dot_general(wn2_ref[...], hh, (((1,), (0,)), ((), ())),
                                   preferred_element_type=f32) + bn2_ref[...]
        mask = mask_ref[...]

        agg = jnp.zeros((cmidp, mpad), f32)
        for i, o in enumerate(offs):
            sl = jax.lax.slice(gp, (0, _MARG + o), (cmidp, _MARG + o + mpad))
            agg = agg + (sl * wtab[:, i:i + 1]) * mask[i:i + 1, :]
        agg = agg * (1.0 / _NBHD)

        out = jax.nn.relu(
            jax.lax.dot_general(w2_ref[...], agg, (((1,), (0,)), ((), ())),
                                preferred_element_type=f32) + b2_ref[...])
        vq = jax.lax.slice(vals, (0, 0), (vals.shape[0], mpad))
        sc = jax.lax.dot_general(wsc_ref[...], vq, (((1,), (0,)), ((), ())),
                                 preferred_element_type=f32) + bsc_ref[...]
        vals = out + sc

    wf_ref, bf_ref, out_ref = refs[pos], refs[pos + 1], refs[pos + 2]
    m_last = dims[-1][4]
    qb = vals.shape[1]
    lane = jax.lax.broadcasted_iota(jnp.int32, (1, qb), 1)
    fmask = (lane < m_last).astype(f32)
    pooled = jnp.sum(vals * fmask, axis=1, keepdims=True) * (1.0 / m_last)
    logits = jax.lax.dot_general(wf_ref[...], pooled, (((1,), (0,)), ((), ())),
                                 preferred_element_type=f32) + bf_ref[...]
    out_ref[0] = logits


def _pad2(w, r, c):
    return jnp.pad(w, ((0, r - w.shape[0]), (0, c - w.shape[1])))


def _col(b, r):
    return jnp.pad(b, (0, r - b.shape[0])).reshape(r, 1)


def kernel(x, params):
    bs, _, h, w = x.shape
    n0 = h * w
    f32 = jnp.float32

    stencil = _stencil_structure(h, w)
    dims = _layer_dims()

    w0 = params['W0']
    b0 = params['b0']

    xr = x.reshape(bs, 3, n0)
    xp = jnp.pad(xr, ((0, 0), (0, 8 - 3), (0, 0)))

    bcast = lambda b: (0, 0)
    args = [xp, _pad2(w0, 16, 8), _col(b0, 16)]
    in_specs = [pl.BlockSpec((1, 8, n0), lambda b: (b, 0, 0)),
                pl.BlockSpec((16, 8), bcast),
                pl.BlockSpec((16, 1), bcast)]
    offsets = []
    for l, (cin, cout, cmid, n, m) in enumerate(dims):
        p = params['layers'][l]
        w1, b1, wsc, bsc = p['W1'], p['b1'], p['Wsc'], p['bsc']
        cinp = _rup(cin, 8)
        cmidp = _rup(cmid, 8)
        coutp = _rup(cout, 8)

        offs, mask_np, dtab_np = stencil[l]
        offsets.append(offs)
        n_off = len(offs)
        mpad = mask_np.shape[1]

        layer_args = [
            jnp.asarray(mask_np),
            jnp.asarray(dtab_np),
            _pad2(w1, cmidp, cinp),
            _col(b1, cmidp),
            p['Wn1'],
            _col(p['bn1'], 32),
            _pad2(p['Wn2'], cmidp, 32),
            _col(p['bn2'], cmidp),
            _pad2(p['W2'], coutp, cmidp),
            _col(p['b2'], coutp),
            _pad2(wsc, coutp, cinp),
            _col(bsc, coutp),
        ]
        args.extend(layer_args)
        in_specs.extend(
            pl.BlockSpec(a.shape, bcast) for a in layer_args)

    coutp = _rup(int(_CHS[-1]), 8)
    wfp = _pad2(params['Wf'], 16, coutp)
    bfp = _col(params['bf'], 16)
    args.extend([wfp, bfp])
    in_specs.extend([pl.BlockSpec((16, coutp), bcast),
                     pl.BlockSpec((16, 1), bcast)])

    kfn = functools.partial(_fused_kernel, dims=tuple(dims),
                            offsets=tuple(offsets))
    logits = pl.pallas_call(
        kfn,
        grid=(bs,),
        in_specs=in_specs,
        out_specs=pl.BlockSpec((1, 16, 1), lambda b: (b, 0, 0)),
        out_shape=jax.ShapeDtypeStruct((bs, 16, 1), f32),
    )(*args)
    return logits.reshape(bs, 16)[:, :_NUM_CLASSES]


# speedup vs baseline: 198.5985x; 1.0923x over previous
import functools

import numpy as np
import jax
import jax.numpy as jnp
from jax.experimental import pallas as pl

_NUM_CLASSES = 10
_K = 8
_KSIZE = 3.66
_NUM_LAYERS = 6
_TOTAL_DS = 1.0 / 256
_NBHD = int(np.round(_KSIZE ** 2))
_DS = _TOTAL_DS ** (1.0 / _NUM_LAYERS)
_CHS = np.round(
    np.exp(np.linspace(np.log(16), np.log(64 * _K), _NUM_LAYERS + 1))
).astype(int)

_MARG = 256


def _rup(v, k):
    return (v + k - 1) // k * k


def _layer_dims():
    dims = []
    n = 64 * 64
    for l in range(_NUM_LAYERS):
        cin = int(_CHS[l])
        cout = int(_CHS[l + 1])
        cmid = max(cout // 4, 1)
        m = int(round(n * _DS))
        dims.append((cin, cout, cmid, n, m))
        n = m
    return dims


def _round_bf16(x):
    u = x.astype(np.float32).view(np.uint32)
    bias = np.uint32(0x7FFF) + ((u >> np.uint32(16)) & np.uint32(1))
    return ((u + bias) & np.uint32(0xFFFF0000)).view(np.float32)


@functools.lru_cache(maxsize=2)
def _stencil_structure(h, w):
    lin_y = np.linspace(-1.0, 1.0, h).astype(np.float32)
    lin_x = np.linspace(-1.0, 1.0, w).astype(np.float32)
    yy, xx = np.meshgrid(lin_y, lin_x, indexing='ij')
    pts = np.stack([yy.reshape(-1), xx.reshape(-1)], axis=1).astype(np.float32)

    out = []
    for (cin, cout, cmid, n, m) in _layer_dims():
        ct = pts[:n]
        qt = pts[:m]
        qb16 = _round_bf16(qt)
        cb16 = _round_bf16(ct)
        py = (qb16[:, 0:1] * cb16[None, :, 0]).astype(np.float32).reshape(m, n)
        px = (qb16[:, 1:2] * cb16[None, :, 1]).astype(np.float32).reshape(m, n)
        dot = py + px
        cn = (ct[:, 0] * ct[:, 0] + ct[:, 1] * ct[:, 1]).astype(np.float32)
        qn = (qt[:, 0] * qt[:, 0] + qt[:, 1] * qt[:, 1]).astype(np.float32)
        d2 = (qn[:, None] + cn[None, :]) - np.float32(2.0) * dot
        idx = np.argsort(d2, axis=1, kind='stable')[:, :_NBHD]

        off = idx - np.arange(m)[:, None]
        offs = np.unique(off)
        omap = {int(o): i for i, o in enumerate(offs)}
        mpad = _rup(m, 128)
        mask = np.zeros((len(offs), mpad), np.float32)
        dtab = np.zeros((2, len(offs)), np.float32)
        dseen = np.zeros(len(offs), bool)
        for q in range(m):
            for j in range(_NBHD):
                o = int(off[q, j])
                i = omap[o]
                mask[i, q] = 1.0
                if not dseen[i]:
                    dtab[:, i] = ct[idx[q, j]] - qt[q]
                    dseen[i] = True
        out.append((tuple(int(o) for o in offs), mask, dtab))
    return out


def _fused_kernel(*refs, dims, offsets):
    f32 = jnp.float32
    x_ref, w0_ref, b0_ref = refs[0], refs[1], refs[2]
    vals = jax.lax.dot_general(w0_ref[...], x_ref[0], (((1,), (0,)), ((), ())),
                               preferred_element_type=f32) + b0_ref[...]
    pos = 3
    for l, (cin, cout, cmid, n, m) in enumerate(dims):
        (mask_ref, dtab_ref, w1_ref, b1_ref, wn1_ref, bn1_ref, wn2_ref,
         bn2_ref, w2_ref, b2_ref, wsc_ref, bsc_ref) = refs[pos:pos + 12]
        pos += 12
        offs = offsets[l]
        mpad = mask_ref.shape[1]
        npad = vals.shape[1]
        cmidp = w1_ref.shape[0]

        g = jax.nn.relu(
            jax.lax.dot_general(w1_ref[...], vals, (((1,), (0,)), ((), ())),
                                preferred_element_type=f32) + b1_ref[...])
        gp = jnp.concatenate(
            [jnp.zeros((cmidp, _MARG), f32), g, jnp.zeros((cmidp, _MARG), f32)],
            axis=1)

        hh = jax.nn.relu(
            jax.lax.dot_general(wn1_ref[...], dtab_ref[...],
                                (((1,), (0,)), ((), ())),
                                preferred_element_type=f32) + bn1_ref[...])
        wtab = jax.lax.dot_general(wn2_ref[...], hh, (((1,), (0,)), ((), ())),
                                   preferred_element_type=f32) + bn2_ref[...]
        mask = mask_ref[...]

        accs = [None, None, None, None]
        for i, o in enumerate(offs):
            sl = jax.lax.slice(gp, (0, _MARG + o), (cmidp, _MARG + o + mpad))
            t = (sl * wtab[:, i:i + 1]) * mask[i:i + 1, :]
            k = i % 4
            accs[k] = t if accs[k] is None else accs[k] + t
        agg = (accs[0] + accs[1]) + (accs[2] + accs[3])
        agg = agg * (1.0 / _NBHD)

        out = jax.nn.relu(
            jax.lax.dot_general(w2_ref[...], agg, (((1,), (0,)), ((), ())),
                                preferred_element_type=f32) + b2_ref[...])
        vq = jax.lax.slice(vals, (0, 0), (vals.shape[0], mpad))
        sc = jax.lax.dot_general(wsc_ref[...], vq, (((1,), (0,)), ((), ())),
                                 preferred_element_type=f32) + bsc_ref[...]
        vals = out + sc

    wf_ref, bf_ref, out_ref = refs[pos], refs[pos + 1], refs[pos + 2]
    m_last = dims[-1][4]
    qb = vals.shape[1]
    lane = jax.lax.broadcasted_iota(jnp.int32, (1, qb), 1)
    fmask = (lane < m_last).astype(f32)
    pooled = jnp.sum(vals * fmask, axis=1, keepdims=True) * (1.0 / m_last)
    logits = jax.lax.dot_general(wf_ref[...], pooled, (((1,), (0,)), ((), ())),
                                 preferred_element_type=f32) + bf_ref[...]
    out_ref[0] = logits


def _pad2(w, r, c):
    return jnp.pad(w, ((0, r - w.shape[0]), (0, c - w.shape[1])))


def _col(b, r):
    return jnp.pad(b, (0, r - b.shape[0])).reshape(r, 1)


def kernel(x, params):
    bs, _, h, w = x.shape
    n0 = h * w
    f32 = jnp.float32

    stencil = _stencil_structure(h, w)
    dims = _layer_dims()

    w0 = params['W0']
    b0 = params['b0']

    xr = x.reshape(bs, 3, n0)
    xp = jnp.pad(xr, ((0, 0), (0, 8 - 3), (0, 0)))

    bcast = lambda b: (0, 0)
    args = [xp, _pad2(w0, 16, 8), _col(b0, 16)]
    in_specs = [pl.BlockSpec((1, 8, n0), lambda b: (b, 0, 0)),
                pl.BlockSpec((16, 8), bcast),
                pl.BlockSpec((16, 1), bcast)]
    offsets = []
    for l, (cin, cout, cmid, n, m) in enumerate(dims):
        p = params['layers'][l]
        w1, b1, wsc, bsc = p['W1'], p['b1'], p['Wsc'], p['bsc']
        cinp = _rup(cin, 8)
        cmidp = _rup(cmid, 8)
        coutp = _rup(cout, 8)

        offs, mask_np, dtab_np = stencil[l]
        offsets.append(offs)
        n_off = len(offs)
        mpad = mask_np.shape[1]

        layer_args = [
            jnp.asarray(mask_np),
            jnp.asarray(dtab_np),
            _pad2(w1, cmidp, cinp),
            _col(b1, cmidp),
            p['Wn1'],
            _col(p['bn1'], 32),
            _pad2(p['Wn2'], cmidp, 32),
            _col(p['bn2'], cmidp),
            _pad2(p['W2'], coutp, cmidp),
            _col(p['b2'], coutp),
            _pad2(wsc, coutp, cinp),
            _col(bsc, coutp),
        ]
        args.extend(layer_args)
        in_specs.extend(
            pl.BlockSpec(a.shape, bcast) for a in layer_args)

    coutp = _rup(int(_CHS[-1]), 8)
    wfp = _pad2(params['Wf'], 16, coutp)
    bfp = _col(params['bf'], 16)
    args.extend([wfp, bfp])
    in_specs.extend([pl.BlockSpec((16, coutp), bcast),
                     pl.BlockSpec((16, 1), bcast)])

    kfn = functools.partial(_fused_kernel, dims=tuple(dims),
                            offsets=tuple(offsets))
    logits = pl.pallas_call(
        kfn,
        grid=(bs,),
        in_specs=in_specs,
        out_specs=pl.BlockSpec((1, 16, 1), lambda b: (b, 0, 0)),
        out_shape=jax.ShapeDtypeStruct((bs, 16, 1), f32),
    )(*args)
    return logits.reshape(bs, 16)[:, :_NUM_CLASSES]


# speedup vs baseline: 282.0254x; 1.4201x over previous
import functools

import numpy as np
import jax
import jax.numpy as jnp
from jax.experimental import pallas as pl

_NUM_CLASSES = 10
_K = 8
_KSIZE = 3.66
_NUM_LAYERS = 6
_TOTAL_DS = 1.0 / 256
_NBHD = int(np.round(_KSIZE ** 2))
_DS = _TOTAL_DS ** (1.0 / _NUM_LAYERS)
_CHS = np.round(
    np.exp(np.linspace(np.log(16), np.log(64 * _K), _NUM_LAYERS + 1))
).astype(int)

_MARG = 256


def _rup(v, k):
    return (v + k - 1) // k * k


def _layer_dims():
    dims = []
    n = 64 * 64
    for l in range(_NUM_LAYERS):
        cin = int(_CHS[l])
        cout = int(_CHS[l + 1])
        cmid = max(cout // 4, 1)
        m = int(round(n * _DS))
        dims.append((cin, cout, cmid, n, m))
        n = m
    return dims


def _round_bf16(x):
    u = x.astype(np.float32).view(np.uint32)
    bias = np.uint32(0x7FFF) + ((u >> np.uint32(16)) & np.uint32(1))
    return ((u + bias) & np.uint32(0xFFFF0000)).view(np.float32)


@functools.lru_cache(maxsize=2)
def _stencil_structure(h, w):
    lin_y = np.linspace(-1.0, 1.0, h).astype(np.float32)
    lin_x = np.linspace(-1.0, 1.0, w).astype(np.float32)
    yy, xx = np.meshgrid(lin_y, lin_x, indexing='ij')
    pts = np.stack([yy.reshape(-1), xx.reshape(-1)], axis=1).astype(np.float32)

    out = []
    for (cin, cout, cmid, n, m) in _layer_dims():
        ct = pts[:n]
        qt = pts[:m]
        qb16 = _round_bf16(qt)
        cb16 = _round_bf16(ct)
        py = (qb16[:, 0:1] * cb16[None, :, 0]).astype(np.float32).reshape(m, n)
        px = (qb16[:, 1:2] * cb16[None, :, 1]).astype(np.float32).reshape(m, n)
        dot = py + px
        cn = (ct[:, 0] * ct[:, 0] + ct[:, 1] * ct[:, 1]).astype(np.float32)
        qn = (qt[:, 0] * qt[:, 0] + qt[:, 1] * qt[:, 1]).astype(np.float32)
        d2 = (qn[:, None] + cn[None, :]) - np.float32(2.0) * dot
        idx = np.argsort(d2, axis=1, kind='stable')[:, :_NBHD]

        off = idx - np.arange(m)[:, None]
        offs = np.unique(off)
        omap = {int(o): i for i, o in enumerate(offs)}
        mpad = _rup(m, 128)
        mask = np.zeros((len(offs), mpad), np.float32)
        dtab = np.zeros((2, len(offs)), np.float32)
        dseen = np.zeros(len(offs), bool)
        for q in range(m):
            for j in range(_NBHD):
                o = int(off[q, j])
                i = omap[o]
                mask[i, q] = 1.0
                if not dseen[i]:
                    dtab[:, i] = ct[idx[q, j]] - qt[q]
                    dseen[i] = True
        out.append((tuple(int(o) for o in offs), mask, dtab))
    return out


def _fused_kernel(*refs, dims, offsets, bs):
    f32 = jnp.float32
    x_ref, w0_ref, b0_ref = refs[0], refs[1], refs[2]
    vals = jax.lax.dot_general(w0_ref[...], x_ref[...],
                               (((1,), (0,)), ((), ())),
                               preferred_element_type=f32) + b0_ref[...]
    pos = 3
    for l, (cin, cout, cmid, n, m) in enumerate(dims):
        (mask_ref, dtab_ref, w1_ref, b1_ref, wn1_ref, bn1_ref, wn2_ref,
         bn2_ref, w2_ref, b2_ref, wsc_ref, bsc_ref) = refs[pos:pos + 12]
        pos += 12
        offs = offsets[l]
        wid = mask_ref.shape[1]
        npad = wid // bs
        mpad = _rup(m, 128)
        cmidp = w1_ref.shape[0]
        cinp = vals.shape[0]

        g = jax.nn.relu(
            jax.lax.dot_general(w1_ref[...], vals, (((1,), (0,)), ((), ())),
                                preferred_element_type=f32) + b1_ref[...])
        gp = jnp.concatenate(
            [jnp.zeros((cmidp, _MARG), f32), g, jnp.zeros((cmidp, _MARG), f32)],
            axis=1)

        hh = jax.nn.relu(
            jax.lax.dot_general(wn1_ref[...], dtab_ref[...],
                                (((1,), (0,)), ((), ())),
                                preferred_element_type=f32) + bn1_ref[...])
        wtab = jax.lax.dot_general(wn2_ref[...], hh, (((1,), (0,)), ((), ())),
                                   preferred_element_type=f32) + bn2_ref[...]
        mask = mask_ref[...]

        accs = [None, None, None, None]
        for i, o in enumerate(offs):
            sl = jax.lax.slice(gp, (0, _MARG + o), (cmidp, _MARG + o + wid))
            t = (sl * wtab[:, i:i + 1]) * mask[i:i + 1, :]
            k = i % 4
            accs[k] = t if accs[k] is None else accs[k] + t
        agg = (accs[0] + accs[1]) + (accs[2] + accs[3])
        agg = agg * (1.0 / _NBHD)

        if mpad != npad:
            aggc = jnp.concatenate(
                [jax.lax.slice(agg, (0, b * npad), (cmidp, b * npad + mpad))
                 for b in range(bs)], axis=1)
            vq = jnp.concatenate(
                [jax.lax.slice(vals, (0, b * npad), (cinp, b * npad + mpad))
                 for b in range(bs)], axis=1)
        else:
            aggc = agg
            vq = vals

        out = jax.nn.relu(
            jax.lax.dot_general(w2_ref[...], aggc, (((1,), (0,)), ((), ())),
                                preferred_element_type=f32) + b2_ref[...])
        sc = jax.lax.dot_general(wsc_ref[...], vq, (((1,), (0,)), ((), ())),
                                 preferred_element_type=f32) + bsc_ref[...]
        vals = out + sc

    wf_ref, bf_ref, out_ref = refs[pos], refs[pos + 1], refs[pos + 2]
    m_last = dims[-1][4]
    mpad = vals.shape[1] // bs
    coutp = vals.shape[0]
    lane = jax.lax.broadcasted_iota(jnp.int32, (1, mpad), 1)
    fmask = (lane < m_last).astype(f32)
    for b in range(bs):
        vb = jax.lax.slice(vals, (0, b * mpad), (coutp, b * mpad + mpad))
        pooled = jnp.sum(vb * fmask, axis=1, keepdims=True) * (1.0 / m_last)
        logits = jax.lax.dot_general(wf_ref[...], pooled,
                                     (((1,), (0,)), ((), ())),
                                     preferred_element_type=f32) + bf_ref[...]
        out_ref[b] = logits


def _pad2(w, r, c):
    return jnp.pad(w, ((0, r - w.shape[0]), (0, c - w.shape[1])))


def _col(b, r):
    return jnp.pad(b, (0, r - b.shape[0])).reshape(r, 1)


def kernel(x, params):
    bs, _, h, w = x.shape
    n0 = h * w
    f32 = jnp.float32

    stencil = _stencil_structure(h, w)
    dims = _layer_dims()

    xr = x.reshape(bs, 3, n0)
    xp = jnp.pad(xr, ((0, 0), (0, 8 - 3), (0, 0)))
    xt = jnp.transpose(xp, (1, 0, 2)).reshape(8, bs * n0)

    args = [xt, _pad2(params['W0'], 16, 8), _col(params['b0'], 16)]
    offsets = []
    npad = n0
    for l, (cin, cout, cmid, n, m) in enumerate(dims):
        p = params['layers'][l]
        cinp = _rup(cin, 8)
        cmidp = _rup(cmid, 8)
        coutp = _rup(cout, 8)

        offs, mask_np, dtab_np = stencil[l]
        offsets.append(offs)
        n_off = len(offs)
        mpad = mask_np.shape[1]

        mask_all = np.zeros((n_off, bs * npad), np.float32)
        for b in range(bs):
            mask_all[:, b * npad:b * npad + mpad] = mask_np

        args.extend([
            jnp.asarray(mask_all),
            jnp.asarray(dtab_np),
            _pad2(p['W1'], cmidp, cinp),
            _col(p['b1'], cmidp),
            p['Wn1'],
            _col(p['bn1'], 32),
            _pad2(p['Wn2'], cmidp, 32),
            _col(p['bn2'], cmidp),
            _pad2(p['W2'], coutp, cmidp),
            _col(p['b2'], coutp),
            _pad2(p['Wsc'], coutp, cinp),
            _col(p['bsc'], coutp),
        ])
        npad = mpad

    coutp = _rup(int(_CHS[-1]), 8)
    args.extend([_pad2(params['Wf'], 16, coutp), _col(params['bf'], 16)])

    kfn = functools.partial(_fused_kernel, dims=tuple(dims),
                            offsets=tuple(offsets), bs=bs)
    logits = pl.pallas_call(
        kfn,
        out_shape=jax.ShapeDtypeStruct((bs, 16, 1), f32),
    )(*args)
    return logits.reshape(bs, 16)[:, :_NUM_CLASSES]


# speedup vs baseline: 346.1291x; 1.2273x over previous
import functools

import numpy as np
import jax
import jax.numpy as jnp
from jax.experimental import pallas as pl

_NUM_CLASSES = 10
_K = 8
_KSIZE = 3.66
_NUM_LAYERS = 6
_TOTAL_DS = 1.0 / 256
_NBHD = int(np.round(_KSIZE ** 2))
_DS = _TOTAL_DS ** (1.0 / _NUM_LAYERS)
_CHS = np.round(
    np.exp(np.linspace(np.log(16), np.log(64 * _K), _NUM_LAYERS + 1))
).astype(int)

_MARG = 256


def _rup(v, k):
    return (v + k - 1) // k * k


def _layer_dims():
    dims = []
    n = 64 * 64
    for l in range(_NUM_LAYERS):
        cin = int(_CHS[l])
        cout = int(_CHS[l + 1])
        cmid = max(cout // 4, 1)
        m = int(round(n * _DS))
        dims.append((cin, cout, cmid, n, m))
        n = m
    return dims


def _round_bf16(x):
    u = x.astype(np.float32).view(np.uint32)
    bias = np.uint32(0x7FFF) + ((u >> np.uint32(16)) & np.uint32(1))
    return ((u + bias) & np.uint32(0xFFFF0000)).view(np.float32)


@functools.lru_cache(maxsize=2)
def _stencil_structure(h, w):
    lin_y = np.linspace(-1.0, 1.0, h).astype(np.float32)
    lin_x = np.linspace(-1.0, 1.0, w).astype(np.float32)
    yy, xx = np.meshgrid(lin_y, lin_x, indexing='ij')
    pts = np.stack([yy.reshape(-1), xx.reshape(-1)], axis=1).astype(np.float32)

    out = []
    for (cin, cout, cmid, n, m) in _layer_dims():
        ct = pts[:n]
        qt = pts[:m]
        qb16 = _round_bf16(qt)
        cb16 = _round_bf16(ct)
        py = (qb16[:, 0:1] * cb16[None, :, 0]).astype(np.float32).reshape(m, n)
        px = (qb16[:, 1:2] * cb16[None, :, 1]).astype(np.float32).reshape(m, n)
        dot = py + px
        cn = (ct[:, 0] * ct[:, 0] + ct[:, 1] * ct[:, 1]).astype(np.float32)
        qn = (qt[:, 0] * qt[:, 0] + qt[:, 1] * qt[:, 1]).astype(np.float32)
        d2 = (qn[:, None] + cn[None, :]) - np.float32(2.0) * dot
        idx = np.argsort(d2, axis=1, kind='stable')[:, :_NBHD]

        off = idx - np.arange(m)[:, None]
        offs = np.unique(off)
        omap = {int(o): i for i, o in enumerate(offs)}
        mpad = _rup(m, 128)
        mask = np.zeros((len(offs), mpad), np.float32)
        dtab = np.zeros((2, len(offs)), np.float32)
        dseen = np.zeros(len(offs), bool)
        for q in range(m):
            for j in range(_NBHD):
                o = int(off[q, j])
                i = omap[o]
                mask[i, q] = 1.0
                if not dseen[i]:
                    dtab[:, i] = ct[idx[q, j]] - qt[q]
                    dseen[i] = True
        out.append((tuple(int(o) for o in offs), mask, dtab))
    return out


def _widths():
    dims = _layer_dims()
    qs = [0] * _NUM_LAYERS
    ws = [0] * _NUM_LAYERS
    qs[-1] = _rup(dims[-1][4], 128)
    for l in range(_NUM_LAYERS - 1, -1, -1):
        (cin, cout, cmid, n, m) = dims[l]
        ws[l] = _rup(min(n, min(m, qs[l]) - 1 + 194), 128)
        if l > 0:
            qs[l - 1] = ws[l]
    return ws, qs


def _fused_kernel(*refs, dims, offsets, bs, qs):
    f32 = jnp.float32
    x_ref, w0_ref, b0_ref = refs[0], refs[1], refs[2]
    vals = jax.lax.dot_general(w0_ref[...], x_ref[...],
                               (((1,), (0,)), ((), ())),
                               preferred_element_type=f32) + b0_ref[...]
    pos = 3
    for l, (cin, cout, cmid, n, m) in enumerate(dims):
        (mask_ref, dtab_ref, w1_ref, b1_ref, wn1_ref, bn1_ref, wn2_ref,
         bn2_ref, w2_ref, b2_ref, wsc_ref, bsc_ref) = refs[pos:pos + 12]
        pos += 12
        offs = offsets[l]
        wid = mask_ref.shape[1]
        npad = wid // bs
        mpad = qs[l]
        cmidp = w1_ref.shape[0]
        cinp = vals.shape[0]

        g = jax.nn.relu(
            jax.lax.dot_general(w1_ref[...], vals, (((1,), (0,)), ((), ())),
                                preferred_element_type=f32) + b1_ref[...])
        gp = jnp.concatenate(
            [jnp.zeros((cmidp, _MARG), f32), g, jnp.zeros((cmidp, _MARG), f32)],
            axis=1)

        hh = jax.nn.relu(
            jax.lax.dot_general(wn1_ref[...], dtab_ref[...],
                                (((1,), (0,)), ((), ())),
                                preferred_element_type=f32) + bn1_ref[...])
        wtab = jax.lax.dot_general(wn2_ref[...], hh, (((1,), (0,)), ((), ())),
                                   preferred_element_type=f32) + bn2_ref[...]
        mask = mask_ref[...]

        accs = [None, None, None, None]
        for i, o in enumerate(offs):
            sl = jax.lax.slice(gp, (0, _MARG + o), (cmidp, _MARG + o + wid))
            t = (sl * wtab[:, i:i + 1]) * mask[i:i + 1, :]
            k = i % 4
            accs[k] = t if accs[k] is None else accs[k] + t
        agg = (accs[0] + accs[1]) + (accs[2] + accs[3])
        agg = agg * (1.0 / _NBHD)

        if mpad != npad:
            aggc = jnp.concatenate(
                [jax.lax.slice(agg, (0, b * npad), (cmidp, b * npad + mpad))
                 for b in range(bs)], axis=1)
            vq = jnp.concatenate(
                [jax.lax.slice(vals, (0, b * npad), (cinp, b * npad + mpad))
                 for b in range(bs)], axis=1)
        else:
            aggc = agg
            vq = vals

        out = jax.nn.relu(
            jax.lax.dot_general(w2_ref[...], aggc, (((1,), (0,)), ((), ())),
                                preferred_element_type=f32) + b2_ref[...])
        sc = jax.lax.dot_general(wsc_ref[...], vq, (((1,), (0,)), ((), ())),
                                 preferred_element_type=f32) + bsc_ref[...]
        vals = out + sc

    wf_ref, bf_ref, out_ref = refs[pos], refs[pos + 1], refs[pos + 2]
    m_last = dims[-1][4]
    mpad = vals.shape[1] // bs
    coutp = vals.shape[0]
    lane = jax.lax.broadcasted_iota(jnp.int32, (1, mpad), 1)
    fmask = (lane < m_last).astype(f32)
    for b in range(bs):
        vb = jax.lax.slice(vals, (0, b * mpad), (coutp, b * mpad + mpad))
        pooled = jnp.sum(vb * fmask, axis=1, keepdims=True) * (1.0 / m_last)
        logits = jax.lax.dot_general(wf_ref[...], pooled,
                                     (((1,), (0,)), ((), ())),
                                     preferred_element_type=f32) + bf_ref[...]
        out_ref[b] = logits


def _pad2(w, r, c):
    return jnp.pad(w, ((0, r - w.shape[0]), (0, c - w.shape[1])))


def _col(b, r):
    return jnp.pad(b, (0, r - b.shape[0])).reshape(r, 1)


def kernel(x, params):
    bs, _, h, w = x.shape
    n0 = h * w
    f32 = jnp.float32

    stencil = _stencil_structure(h, w)
    dims = _layer_dims()

    ws_l, qs_l = _widths()
    xr = x.reshape(bs, 3, n0)[:, :, :ws_l[0]]
    xp = jnp.pad(xr, ((0, 0), (0, 8 - 3), (0, 0)))
    xt = jnp.transpose(xp, (1, 0, 2)).reshape(8, bs * ws_l[0])

    args = [xt, _pad2(params['W0'], 16, 8), _col(params['b0'], 16)]
    offsets = []
    for l, (cin, cout, cmid, n, m) in enumerate(dims):
        p = params['layers'][l]
        cinp = _rup(cin, 8)
        cmidp = _rup(cmid, 8)
        coutp = _rup(cout, 8)

        offs, mask_np, dtab_np = stencil[l]
        offsets.append(offs)
        n_off = len(offs)
        wl = ws_l[l]
        keep = min(qs_l[l], mask_np.shape[1])

        mask_all = np.zeros((n_off, bs * wl), np.float32)
        for b in range(bs):
            mask_all[:, b * wl:b * wl + keep] = mask_np[:, :keep]

        args.extend([
            jnp.asarray(mask_all),
            jnp.asarray(dtab_np),
            _pad2(p['W1'], cmidp, cinp),
            _col(p['b1'], cmidp),
            p['Wn1'],
            _col(p['bn1'], 32),
            _pad2(p['Wn2'], cmidp, 32),
            _col(p['bn2'], cmidp),
            _pad2(p['W2'], coutp, cmidp),
            _col(p['b2'], coutp),
            _pad2(p['Wsc'], coutp, cinp),
            _col(p['bsc'], coutp),
        ])

    coutp = _rup(int(_CHS[-1]), 8)
    args.extend([_pad2(params['Wf'], 16, coutp), _col(params['bf'], 16)])

    kfn = functools.partial(_fused_kernel, dims=tuple(dims),
                            offsets=tuple(offsets), bs=bs,
                            qs=tuple(qs_l))
    logits = pl.pallas_call(
        kfn,
        out_shape=jax.ShapeDtypeStruct((bs, 16, 1), f32),
    )(*args)
    return logits.reshape(bs, 16)[:, :_NUM_CLASSES]
